# baseline (speedup 1.0000x reference)
import functools

import jax
import jax.numpy as jnp
from jax import lax
from jax.experimental import pallas as pl
from jax.experimental.pallas import tpu as pltpu
from jax.experimental.pallas import tpu_sc as plsc

HIDDEN = 768
INTER = 3072
E = 8
TOP_K = 2

TN = 512
CAP0 = 768
CAP1 = 2048 - CAP0


def _ffn_body(tok_ref, w_ref, x2_ref, gu_g_ref, gu_u_ref, dw_ref, out_ref,
              xs_ref):
    j = pl.program_id(1)

    @pl.when(j == 0)
    def _gather():
        tokc = tok_ref[...]
        lanes = jax.lax.broadcasted_iota(jnp.int32, (tokc.shape[0], x2_ref.shape[0]), 1)
        p = (lanes == tokc).astype(jnp.bfloat16)
        xs_ref[...] = jax.lax.dot_general(
            p, x2_ref[...].astype(jnp.bfloat16), (((1,), (0,)), ((), ())),
            preferred_element_type=jnp.float32).astype(jnp.bfloat16)

    xb = xs_ref[...]
    hg = jax.lax.dot_general(xb, gu_g_ref[0].astype(jnp.bfloat16),
                             (((1,), (1,)), ((), ())),
                             preferred_element_type=jnp.float32)
    hu = jax.lax.dot_general(xb, gu_u_ref[0].astype(jnp.bfloat16),
                             (((1,), (1,)), ((), ())),
                             preferred_element_type=jnp.float32)
    act = (hg * jax.nn.sigmoid(hg)) * hu * w_ref[...]
    part = jax.lax.dot_general(act.astype(jnp.bfloat16),
                               dw_ref[0].astype(jnp.bfloat16),
                               (((1,), (1,)), ((), ())),
                               preferred_element_type=jnp.float32)

    @pl.when(j == 0)
    def _init():
        out_ref[...] = part

    @pl.when(j != 0)
    def _acc():
        out_ref[...] += part


def _grouped_ffn(tok_buf, wslot, x2, gu_w, down_w, cap):
    nt = INTER // TN
    s = x2.shape[0]
    return pl.pallas_call(
        _ffn_body,
        grid=(E, nt),
        in_specs=[
            pl.BlockSpec((cap, 1), lambda e, j: (e, 0)),
            pl.BlockSpec((cap, 1), lambda e, j: (e, 0)),
            pl.BlockSpec((s, HIDDEN), lambda e, j: (0, 0)),
            pl.BlockSpec((1, TN, HIDDEN), lambda e, j: (e, j, 0)),
            pl.BlockSpec((1, TN, HIDDEN), lambda e, j: (e, nt + j, 0)),
            pl.BlockSpec((1, HIDDEN, TN), lambda e, j: (e, 0, j)),
        ],
        out_specs=pl.BlockSpec((cap, HIDDEN), lambda e, j: (e, 0)),
        out_shape=jax.ShapeDtypeStruct((E * cap, HIDDEN), jnp.float32),
        scratch_shapes=[pltpu.VMEM((cap, HIDDEN), jnp.bfloat16)],
    )(tok_buf, wslot, x2, gu_w, gu_w, down_w)


def _sc_combine(ys, idx_a, idx_b, chunk):
    S, = idx_a.shape
    D = ys.shape[1]
    info = plsc.get_sparse_core_info()
    nc, ns = info.num_cores, info.num_subcores
    nw = nc * ns
    per_w = S // nw
    nch = per_w // chunk
    mesh = plsc.VectorSubcoreMesh(core_axis_name="c", subcore_axis_name="s")

    @functools.partial(
        pl.kernel, mesh=mesh,
        out_type=jax.ShapeDtypeStruct((S, D), jnp.float32),
        scratch_types=[
            pltpu.VMEM((chunk,), jnp.int32),
            pltpu.VMEM((chunk,), jnp.int32),
            pltpu.VMEM((chunk, D), jnp.float32),
            pltpu.VMEM((chunk, D), jnp.float32),
            pltpu.SemaphoreType.DMA,
            pltpu.SemaphoreType.DMA,
        ],
    )
    def k(ys_hbm, ia_hbm, ib_hbm, out_hbm, ia_v, ib_v, ra_v, rb_v, sem, sem2):
        wid = lax.axis_index("s") * nc + lax.axis_index("c")
        base = wid * per_w
        for c in range(nch):
            off = base + c * chunk
            pltpu.sync_copy(ia_hbm.at[pl.ds(off, chunk)], ia_v)
            pltpu.sync_copy(ib_hbm.at[pl.ds(off, chunk)], ib_v)
            cp_a = pltpu.async_copy(ys_hbm.at[ia_v], ra_v, sem)
            cp_b = pltpu.async_copy(ys_hbm.at[ib_v], rb_v, sem2)
            cp_a.wait()
            cp_b.wait()

            def add_row(t, _):
                for d in range(D // 16):
                    sl = pl.ds(d * 16, 16)
                    ra_v[t, sl] = ra_v[t, sl] + rb_v[t, sl]
                return 0

            lax.fori_loop(0, chunk, add_row, 0)
            pltpu.sync_copy(ra_v, out_hbm.at[pl.ds(off, chunk)])

    return k(ys, idx_a, idx_b)


def kernel(x, gate_w, gu_w, down_w):
    b, s, h = x.shape
    x2 = x.reshape(s, h)

    logits = jnp.einsum('sh,eh->se', x2, gate_w)
    probs = jax.nn.softmax(logits, axis=-1)
    topv, topi = jax.lax.top_k(probs, TOP_K)
    denom = jnp.clip(jnp.sum(topv, axis=-1, keepdims=True), 1e-9, None)
    wgt = (topv / denom).reshape(-1)
    eid = topi.reshape(-1).astype(jnp.int32)
    tok = (jnp.arange(2 * s, dtype=jnp.int32) // 2)

    onehot = (eid[:, None] == jnp.arange(E, dtype=jnp.int32)[None, :]).astype(jnp.int32)
    rank = jnp.take_along_axis(jnp.cumsum(onehot, axis=0), eid[:, None], 1)[:, 0] - 1
    in0 = rank < CAP0
    m0 = E * CAP0
    slot0 = eid * CAP0 + rank
    slot0_c = eid * CAP0 + jnp.minimum(rank, CAP0 - 1)

    scat0 = jnp.where(in0, slot0, m0)
    tok_buf0 = jnp.zeros((m0,), jnp.int32).at[scat0].set(tok, mode='drop')
    w_buf0 = jnp.zeros((m0,), jnp.float32).at[scat0].set(wgt, mode='drop')

    ys0 = _grouped_ffn(tok_buf0.reshape(m0, 1), w_buf0.reshape(m0, 1),
                       x2, gu_w, down_w, CAP0)

    out_fast = _sc_combine(ys0, slot0_c[0::2], slot0_c[1::2], 32)

    def _slow(_):
        g0 = ys0[jnp.where(in0, slot0, 0)] * in0[:, None].astype(jnp.float32)
        m1 = E * CAP1
        slot1 = eid * CAP1 + (rank - CAP0)
        scat1 = jnp.where(in0, m1, slot1)
        tok_buf1 = jnp.zeros((m1,), jnp.int32).at[scat1].set(tok, mode='drop')
        w_buf1 = jnp.zeros((m1,), jnp.float32).at[scat1].set(wgt, mode='drop')
        ys1 = _grouped_ffn(tok_buf1.reshape(m1, 1), w_buf1.reshape(m1, 1),
                           x2, gu_w, down_w, CAP1)
        g1 = ys1[jnp.where(in0, 0, slot1)] * (~in0)[:, None].astype(jnp.float32)
        return (g0 + g1).reshape(s, TOP_K, h).sum(axis=1)

    out = jax.lax.cond(jnp.any(~in0), _slow, lambda _: out_fast, 0)
    return out.reshape(b, s, h)
